# baseline (speedup 1.0000x reference)
import jax
import jax.numpy as jnp
from jax import lax
from jax.experimental import pallas as pl
from jax.experimental.pallas import tpu as pltpu
from jax.experimental.pallas import tpu_sc as plsc

N = 10000
E = 320000
NC = 2
NS = 16
NW = NC * NS
CHUNK = 128
NCHUNK = 80
P = CHUNK * NCHUNK
EP = P * NW
NP = 10240
ROWS_PER_TILE = NP // NS


_CS_COLS = 512
_CS_ROWS = EP // _CS_COLS
_CS_BLOCK_ROWS = 8
_CS_GRID = _CS_ROWS // _CS_BLOCK_ROWS


def _cos_sin_body(theta_ref, g_ref, cos_ref, sin_ref):
    i = pl.program_id(0)
    ang = theta_ref[...] + g_ref[...]
    ridx = lax.broadcasted_iota(jnp.int32, (_CS_BLOCK_ROWS, _CS_COLS), 0)
    cidx = lax.broadcasted_iota(jnp.int32, (_CS_BLOCK_ROWS, _CS_COLS), 1)
    flat = (i * _CS_BLOCK_ROWS + ridx) * _CS_COLS + cidx
    valid = flat < E
    cos_ref[...] = jnp.where(valid, jnp.cos(ang), 0.0)
    sin_ref[...] = jnp.where(valid, jnp.sin(ang), 0.0)


def _cos_sin(theta_p, g_p):
    spec = pl.BlockSpec((_CS_BLOCK_ROWS, _CS_COLS), lambda i: (i, 0))
    out = pl.pallas_call(
        _cos_sin_body,
        grid=(_CS_GRID,),
        in_specs=[spec, spec],
        out_specs=[spec, spec],
        out_shape=[
            jax.ShapeDtypeStruct((_CS_ROWS, _CS_COLS), jnp.float32),
            jax.ShapeDtypeStruct((_CS_ROWS, _CS_COLS), jnp.float32),
        ],
    )(theta_p.reshape(_CS_ROWS, _CS_COLS), g_p.reshape(_CS_ROWS, _CS_COLS))
    return out[0], out[1]


_MM_ROWS = 512
_MM_GRID = NP // _MM_ROWS


def _mm_body(h_ref, wself_ref, wcs_ref, self_ref, table_ref):
    h = h_ref[...]
    self_ref[...] = jnp.dot(h, wself_ref[...], preferred_element_type=jnp.float32)
    tab = jnp.dot(h, wcs_ref[...], preferred_element_type=jnp.float32)
    table_ref[...] = tab.astype(jnp.bfloat16)


def _row_spec(cols):
    return pl.BlockSpec((_MM_ROWS, cols), lambda i: (i, 0))


def _full_spec(r, c):
    return pl.BlockSpec((r, c), lambda i: (0, 0))


def _mm(h, w_self, w_cs):
    din = h.shape[1]
    dself = w_self.shape[1]
    dtab = w_cs.shape[1]
    return pl.pallas_call(
        _mm_body,
        grid=(_MM_GRID,),
        in_specs=[_row_spec(din), _full_spec(din, dself), _full_spec(din, dtab)],
        out_specs=[_row_spec(dself), _row_spec(dtab)],
        out_shape=[
            jax.ShapeDtypeStruct((NP, dself), jnp.float32),
            jax.ShapeDtypeStruct((NP, dtab), jnp.bfloat16),
        ],
    )(h, w_self, w_cs)


def _combine_body(selfp_ref, a00_ref, a01_ref, a10_ref, a11_ref, out_ref):
    lo = a00_ref[...] + a01_ref[...]
    hi = a10_ref[...] + a11_ref[...]
    agg = jnp.concatenate([lo, hi], axis=1)
    out_ref[...] = jnp.maximum(selfp_ref[...] + agg, 0.0)


def _combine(self_i, aggs):
    return pl.pallas_call(
        _combine_body,
        grid=(_MM_GRID,),
        in_specs=[_row_spec(128), _row_spec(64), _row_spec(64),
                  _row_spec(64), _row_spec(64)],
        out_specs=_row_spec(128),
        out_shape=jax.ShapeDtypeStruct((NP, 128), jnp.float32),
    )(self_i, aggs[0, 0], aggs[0, 1], aggs[1, 0], aggs[1, 1])


def _make_edge_kernel(dout):
    nred = dout // 16
    mesh = plsc.VectorSubcoreMesh(core_axis_name="c", subcore_axis_name="s")

    def body(table_hbm, src_hbm, src2_hbm, dst_hbm, cos_hbm, sin_hbm,
             zeros_hbm, out_hbm,
             srcv, srcv2, dstv, cosv, sinv, rows0, rows1, msg, agg, sem0, sem1):
        cid = lax.axis_index("c")
        sid = lax.axis_index("s")
        wid = sid * NC + cid

        pltpu.sync_copy(src_hbm.at[wid], srcv)
        pltpu.sync_copy(src2_hbm.at[wid], srcv2)
        pltpu.sync_copy(dst_hbm.at[wid], dstv)
        pltpu.sync_copy(cos_hbm.at[wid], cosv)
        pltpu.sync_copy(sin_hbm.at[wid], sinv)

        himask = jnp.int32(-65536)

        def compute_scatter(j, rows):
            @plsc.parallel_loop(0, CHUNK // 16, unroll=4)
            def edge_body(q):
                cvec = cosv[j, pl.ds(q * 16, 16)]
                svec = sinv[j, pl.ds(q * 16, 16)]
                for i in range(16):
                    e = q * 16 + i
                    c = cvec[i]
                    s = svec[i]
                    for k in range(dout // 32):
                        wa = rows[e, pl.ds(16 * k, 16)]
                        wb = rows[e, pl.ds(dout // 2 + 16 * k, 16)]
                        a_lo = lax.bitcast_convert_type(wa << 16, jnp.float32)
                        a_hi = lax.bitcast_convert_type(wa & himask, jnp.float32)
                        b_lo = lax.bitcast_convert_type(wb << 16, jnp.float32)
                        b_hi = lax.bitcast_convert_type(wb & himask, jnp.float32)
                        msg[e, pl.ds(32 * k, 16)] = c * a_lo + s * b_lo
                        msg[e, pl.ds(32 * k + 16, 16)] = c * a_hi + s * b_hi
            pltpu.sync_copy(msg, agg.at[dstv.at[j]], add=True)

        for p, idxv in ((0, srcv), (1, srcv2)):
            pltpu.sync_copy(
                zeros_hbm.at[pl.ds(sid * ROWS_PER_TILE, ROWS_PER_TILE)],
                agg.at[pl.ds(sid * ROWS_PER_TILE, ROWS_PER_TILE)])
            plsc.subcore_barrier()

            pltpu.async_copy(table_hbm.at[idxv.at[0]], rows0, sem0)

            def chunk_pair(i, carry):
                j0 = 2 * i
                pltpu.async_copy(table_hbm.at[idxv.at[j0 + 1]], rows1, sem1)
                pltpu.make_async_copy(
                    table_hbm.at[idxv.at[j0]], rows0, sem0).wait()
                compute_scatter(j0, rows0)

                @pl.when(j0 + 2 < NCHUNK)
                def _():
                    pltpu.async_copy(table_hbm.at[idxv.at[j0 + 2]], rows0, sem0)

                pltpu.make_async_copy(
                    table_hbm.at[idxv.at[j0 + 1]], rows1, sem1).wait()
                compute_scatter(j0 + 1, rows1)
                return carry

            lax.fori_loop(0, NCHUNK // 2, chunk_pair, 0)
            plsc.subcore_barrier()

            pltpu.sync_copy(
                agg.at[pl.ds(sid * ROWS_PER_TILE, ROWS_PER_TILE)],
                out_hbm.at[p, cid, pl.ds(sid * ROWS_PER_TILE, ROWS_PER_TILE)])

    return pl.kernel(
        body,
        out_type=jax.ShapeDtypeStruct((2, NC, NP, dout), jnp.float32),
        mesh=mesh,
        scratch_types=[
            pltpu.VMEM((NCHUNK, CHUNK), jnp.int32),
            pltpu.VMEM((NCHUNK, CHUNK), jnp.int32),
            pltpu.VMEM((NCHUNK, CHUNK), jnp.int32),
            pltpu.VMEM((NCHUNK, CHUNK), jnp.float32),
            pltpu.VMEM((NCHUNK, CHUNK), jnp.float32),
            pltpu.VMEM((CHUNK, dout), jnp.int32),
            pltpu.VMEM((CHUNK, dout), jnp.int32),
            pltpu.VMEM((CHUNK, dout), jnp.float32),
            pltpu.VMEM_SHARED((NP, dout), jnp.float32),
            pltpu.SemaphoreType.DMA,
            pltpu.SemaphoreType.DMA,
        ],
        compiler_params=pltpu.CompilerParams(use_tc_tiling_on_sc=False),
    )


def _pad_rows(w, rows):
    return jnp.concatenate([w, jnp.zeros((rows - w.shape[0], w.shape[1]),
                                         jnp.float32)], axis=0)


def kernel(pos, x, edge_index, theta, g,
           W0_self, W0_c, W0_s, W1_self, W1_c, W1_s, W2_self, W2_c, W2_s):
    h0 = jnp.concatenate([pos, x], axis=1)
    h0 = jnp.concatenate([h0, jnp.zeros((NP - N, 128), jnp.float32)], axis=0)
    src = edge_index[0]
    dst = edge_index[1]

    pad = EP - E
    padi = jnp.zeros((pad,), jnp.int32)
    padf = jnp.zeros((pad,), jnp.float32)
    src_p = jnp.concatenate([src, padi]).reshape(NW, NCHUNK, CHUNK)
    dst_p = jnp.concatenate([dst, padi]).reshape(NW, NCHUNK, CHUNK)
    theta_p = jnp.concatenate([theta, padf])
    g_p = jnp.concatenate([g, padf])

    cos_f, sin_f = _cos_sin(theta_p, g_p)
    cos_p = cos_f.reshape(NW, NCHUNK, CHUNK)
    sin_p = sin_f.reshape(NW, NCHUNK, CHUNK)

    zeros64 = jnp.zeros((NP, 64), jnp.float32)
    edge64 = _make_edge_kernel(64)

    z32 = jnp.zeros((96, 32), jnp.float32)

    perm = []
    for k in (0, 1):
        for i in range(16):
            perm.extend([32 * k + i, 32 * k + 16 + i])
    perm = jnp.array(perm, jnp.int32)

    def cs_pair(wc, ws, lo, hi, width):
        zc = jnp.zeros((wc.shape[0], 64 - width), jnp.float32)
        yc = jnp.concatenate([wc[:, lo:hi], zc], axis=1)[:, perm]
        ys = jnp.concatenate([ws[:, lo:hi], zc], axis=1)[:, perm]
        return jnp.concatenate([yc, ys], axis=1)

    wself = jnp.stack([
        jnp.concatenate([W0_self, jnp.zeros((128, 32), jnp.float32)], axis=1),
        _pad_rows(jnp.concatenate([W1_self, z32], axis=1), 128),
        _pad_rows(W2_self, 128),
    ])
    wcs = jnp.stack([
        jnp.concatenate([cs_pair(W0_c, W0_s, 0, 64, 64),
                         cs_pair(W0_c, W0_s, 64, 96, 32)], axis=1),
        _pad_rows(jnp.concatenate([cs_pair(W1_c, W1_s, 0, 64, 64),
                                   cs_pair(W1_c, W1_s, 64, 96, 32)], axis=1),
                  128),
        _pad_rows(jnp.concatenate([cs_pair(W2_c, W2_s, 0, 64, 64),
                                   cs_pair(W2_c, W2_s, 64, 128, 64)], axis=1),
                  128),
    ])

    src_p2 = src_p + NP

    def layer_body(carry, ws):
        h, k = carry
        w_self, w_cs = ws
        self_i, tabflat = _mm(h, w_self, w_cs)
        tab32 = lax.bitcast_convert_type(
            tabflat.reshape(NP, 128, 2), jnp.int32)
        tables = jnp.concatenate([tab32[:, :64], tab32[:, 64:]], axis=0)
        aggs = edge64(tables, src_p, src_p2, dst_p, cos_p, sin_p, zeros64)
        cand = _combine(self_i, aggs)
        h_next = jnp.where(k < 2, cand, h)
        return (h_next, k + 1), cand

    (_, _), cands = lax.scan(layer_body, (h0, jnp.int32(0)), (wself, wcs))
    return cands[2][:N]


# speedup vs baseline: 1.0026x; 1.0026x over previous
import jax
import jax.numpy as jnp
from jax import lax
from jax.experimental import pallas as pl
from jax.experimental.pallas import tpu as pltpu
from jax.experimental.pallas import tpu_sc as plsc

N = 10000
E = 320000
NC = 2
NS = 16
NW = NC * NS
CHUNK = 128
NCHUNK = 80
P = CHUNK * NCHUNK
EP = P * NW
NP = 10240
ROWS_PER_TILE = NP // NS


_CS_COLS = 512
_CS_ROWS = EP // _CS_COLS
_CS_BLOCK_ROWS = 8
_CS_GRID = _CS_ROWS // _CS_BLOCK_ROWS


def _cos_sin_body(theta_ref, g_ref, cos_ref, sin_ref):
    i = pl.program_id(0)
    ang = theta_ref[...] + g_ref[...]
    ridx = lax.broadcasted_iota(jnp.int32, (_CS_BLOCK_ROWS, _CS_COLS), 0)
    cidx = lax.broadcasted_iota(jnp.int32, (_CS_BLOCK_ROWS, _CS_COLS), 1)
    flat = (i * _CS_BLOCK_ROWS + ridx) * _CS_COLS + cidx
    valid = flat < E
    cos_ref[...] = jnp.where(valid, jnp.cos(ang), 0.0)
    sin_ref[...] = jnp.where(valid, jnp.sin(ang), 0.0)


def _cos_sin(theta_p, g_p):
    spec = pl.BlockSpec((_CS_BLOCK_ROWS, _CS_COLS), lambda i: (i, 0))
    out = pl.pallas_call(
        _cos_sin_body,
        grid=(_CS_GRID,),
        in_specs=[spec, spec],
        out_specs=[spec, spec],
        out_shape=[
            jax.ShapeDtypeStruct((_CS_ROWS, _CS_COLS), jnp.float32),
            jax.ShapeDtypeStruct((_CS_ROWS, _CS_COLS), jnp.float32),
        ],
    )(theta_p.reshape(_CS_ROWS, _CS_COLS), g_p.reshape(_CS_ROWS, _CS_COLS))
    return out[0], out[1]


_MM_ROWS = 512
_MM_GRID = NP // _MM_ROWS


def _mm_body(h_ref, wself_ref, wcs_ref, self_ref, table_ref):
    h = h_ref[...]
    self_ref[...] = jnp.dot(h, wself_ref[...], preferred_element_type=jnp.float32)
    tab = jnp.dot(h, wcs_ref[...], preferred_element_type=jnp.float32)
    table_ref[...] = tab.astype(jnp.bfloat16)


def _row_spec(cols):
    return pl.BlockSpec((_MM_ROWS, cols), lambda i: (i, 0))


def _full_spec(r, c):
    return pl.BlockSpec((r, c), lambda i: (0, 0))


def _mm(h, w_self, w_cs):
    din = h.shape[1]
    dself = w_self.shape[1]
    dtab = w_cs.shape[1]
    return pl.pallas_call(
        _mm_body,
        grid=(_MM_GRID,),
        in_specs=[_row_spec(din), _full_spec(din, dself), _full_spec(din, dtab)],
        out_specs=[_row_spec(dself), _row_spec(dtab)],
        out_shape=[
            jax.ShapeDtypeStruct((NP, dself), jnp.float32),
            jax.ShapeDtypeStruct((NP, dtab), jnp.bfloat16),
        ],
    )(h, w_self, w_cs)


def _combine_body(selfp_ref, a00_ref, a01_ref, a10_ref, a11_ref, out_ref):
    lo = a00_ref[...] + a01_ref[...]
    hi = a10_ref[...] + a11_ref[...]
    agg = jnp.concatenate([lo, hi], axis=1)
    out_ref[...] = jnp.maximum(selfp_ref[...] + agg, 0.0)


def _combine(self_i, aggs):
    return pl.pallas_call(
        _combine_body,
        grid=(_MM_GRID,),
        in_specs=[_row_spec(128), _row_spec(64), _row_spec(64),
                  _row_spec(64), _row_spec(64)],
        out_specs=_row_spec(128),
        out_shape=jax.ShapeDtypeStruct((NP, 128), jnp.float32),
    )(self_i, aggs[0, 0], aggs[0, 1], aggs[1, 0], aggs[1, 1])


def _make_edge_kernel(dout):
    nred = dout // 16
    mesh = plsc.VectorSubcoreMesh(core_axis_name="c", subcore_axis_name="s")

    def body(table_hbm, src_hbm, src2_hbm, dst_hbm, cos_hbm, sin_hbm,
             zeros_hbm, out_hbm,
             srcv, srcv2, dstv, cosv, sinv, rows0, rows1, msg, agg, sem0, sem1):
        cid = lax.axis_index("c")
        sid = lax.axis_index("s")
        wid = sid * NC + cid

        pltpu.sync_copy(src_hbm.at[wid], srcv)
        pltpu.sync_copy(src2_hbm.at[wid], srcv2)
        pltpu.sync_copy(dst_hbm.at[wid], dstv)
        pltpu.sync_copy(cos_hbm.at[wid], cosv)
        pltpu.sync_copy(sin_hbm.at[wid], sinv)

        himask = jnp.int32(-65536)

        def compute_scatter(j, rows):
            @plsc.parallel_loop(0, CHUNK // 16, unroll=2)
            def edge_body(q):
                cvec = cosv[j, pl.ds(q * 16, 16)]
                svec = sinv[j, pl.ds(q * 16, 16)]
                for i in range(16):
                    e = q * 16 + i
                    c = cvec[i]
                    s = svec[i]
                    for k in range(dout // 32):
                        wa = rows[e, pl.ds(16 * k, 16)]
                        wb = rows[e, pl.ds(dout // 2 + 16 * k, 16)]
                        a_lo = lax.bitcast_convert_type(wa << 16, jnp.float32)
                        a_hi = lax.bitcast_convert_type(wa & himask, jnp.float32)
                        b_lo = lax.bitcast_convert_type(wb << 16, jnp.float32)
                        b_hi = lax.bitcast_convert_type(wb & himask, jnp.float32)
                        msg[e, pl.ds(32 * k, 16)] = c * a_lo + s * b_lo
                        msg[e, pl.ds(32 * k + 16, 16)] = c * a_hi + s * b_hi
            pltpu.sync_copy(msg, agg.at[dstv.at[j]], add=True)

        for p, idxv in ((0, srcv), (1, srcv2)):
            pltpu.sync_copy(
                zeros_hbm.at[pl.ds(sid * ROWS_PER_TILE, ROWS_PER_TILE)],
                agg.at[pl.ds(sid * ROWS_PER_TILE, ROWS_PER_TILE)])
            plsc.subcore_barrier()

            pltpu.async_copy(table_hbm.at[idxv.at[0]], rows0, sem0)

            def chunk_pair(i, carry):
                j0 = 2 * i
                pltpu.async_copy(table_hbm.at[idxv.at[j0 + 1]], rows1, sem1)
                pltpu.make_async_copy(
                    table_hbm.at[idxv.at[j0]], rows0, sem0).wait()
                compute_scatter(j0, rows0)

                @pl.when(j0 + 2 < NCHUNK)
                def _():
                    pltpu.async_copy(table_hbm.at[idxv.at[j0 + 2]], rows0, sem0)

                pltpu.make_async_copy(
                    table_hbm.at[idxv.at[j0 + 1]], rows1, sem1).wait()
                compute_scatter(j0 + 1, rows1)
                return carry

            lax.fori_loop(0, NCHUNK // 2, chunk_pair, 0)
            plsc.subcore_barrier()

            pltpu.sync_copy(
                agg.at[pl.ds(sid * ROWS_PER_TILE, ROWS_PER_TILE)],
                out_hbm.at[p, cid, pl.ds(sid * ROWS_PER_TILE, ROWS_PER_TILE)])

    return pl.kernel(
        body,
        out_type=jax.ShapeDtypeStruct((2, NC, NP, dout), jnp.float32),
        mesh=mesh,
        scratch_types=[
            pltpu.VMEM((NCHUNK, CHUNK), jnp.int32),
            pltpu.VMEM((NCHUNK, CHUNK), jnp.int32),
            pltpu.VMEM((NCHUNK, CHUNK), jnp.int32),
            pltpu.VMEM((NCHUNK, CHUNK), jnp.float32),
            pltpu.VMEM((NCHUNK, CHUNK), jnp.float32),
            pltpu.VMEM((CHUNK, dout), jnp.int32),
            pltpu.VMEM((CHUNK, dout), jnp.int32),
            pltpu.VMEM((CHUNK, dout), jnp.float32),
            pltpu.VMEM_SHARED((NP, dout), jnp.float32),
            pltpu.SemaphoreType.DMA,
            pltpu.SemaphoreType.DMA,
        ],
        compiler_params=pltpu.CompilerParams(use_tc_tiling_on_sc=False),
    )


def _pad_rows(w, rows):
    return jnp.concatenate([w, jnp.zeros((rows - w.shape[0], w.shape[1]),
                                         jnp.float32)], axis=0)


def kernel(pos, x, edge_index, theta, g,
           W0_self, W0_c, W0_s, W1_self, W1_c, W1_s, W2_self, W2_c, W2_s):
    h0 = jnp.concatenate([pos, x], axis=1)
    h0 = jnp.concatenate([h0, jnp.zeros((NP - N, 128), jnp.float32)], axis=0)
    src = edge_index[0]
    dst = edge_index[1]

    pad = EP - E
    padi = jnp.zeros((pad,), jnp.int32)
    padf = jnp.zeros((pad,), jnp.float32)
    src_p = jnp.concatenate([src, padi]).reshape(NW, NCHUNK, CHUNK)
    dst_p = jnp.concatenate([dst, padi]).reshape(NW, NCHUNK, CHUNK)
    theta_p = jnp.concatenate([theta, padf])
    g_p = jnp.concatenate([g, padf])

    cos_f, sin_f = _cos_sin(theta_p, g_p)
    cos_p = cos_f.reshape(NW, NCHUNK, CHUNK)
    sin_p = sin_f.reshape(NW, NCHUNK, CHUNK)

    zeros64 = jnp.zeros((NP, 64), jnp.float32)
    edge64 = _make_edge_kernel(64)

    z32 = jnp.zeros((96, 32), jnp.float32)

    perm = []
    for k in (0, 1):
        for i in range(16):
            perm.extend([32 * k + i, 32 * k + 16 + i])
    perm = jnp.array(perm, jnp.int32)

    def cs_pair(wc, ws, lo, hi, width):
        zc = jnp.zeros((wc.shape[0], 64 - width), jnp.float32)
        yc = jnp.concatenate([wc[:, lo:hi], zc], axis=1)[:, perm]
        ys = jnp.concatenate([ws[:, lo:hi], zc], axis=1)[:, perm]
        return jnp.concatenate([yc, ys], axis=1)

    wself = jnp.stack([
        jnp.concatenate([W0_self, jnp.zeros((128, 32), jnp.float32)], axis=1),
        _pad_rows(jnp.concatenate([W1_self, z32], axis=1), 128),
        _pad_rows(W2_self, 128),
    ])
    wcs = jnp.stack([
        jnp.concatenate([cs_pair(W0_c, W0_s, 0, 64, 64),
                         cs_pair(W0_c, W0_s, 64, 96, 32)], axis=1),
        _pad_rows(jnp.concatenate([cs_pair(W1_c, W1_s, 0, 64, 64),
                                   cs_pair(W1_c, W1_s, 64, 96, 32)], axis=1),
                  128),
        _pad_rows(jnp.concatenate([cs_pair(W2_c, W2_s, 0, 64, 64),
                                   cs_pair(W2_c, W2_s, 64, 128, 64)], axis=1),
                  128),
    ])

    src_p2 = src_p + NP

    def layer_body(carry, ws):
        h, k = carry
        w_self, w_cs = ws
        self_i, tabflat = _mm(h, w_self, w_cs)
        tab32 = lax.bitcast_convert_type(
            tabflat.reshape(NP, 128, 2), jnp.int32)
        tables = jnp.concatenate([tab32[:, :64], tab32[:, 64:]], axis=0)
        aggs = edge64(tables, src_p, src_p2, dst_p, cos_p, sin_p, zeros64)
        cand = _combine(self_i, aggs)
        h_next = jnp.where(k < 2, cand, h)
        return (h_next, k + 1), cand

    (_, _), cands = lax.scan(layer_body, (h0, jnp.int32(0)), (wself, wcs))
    return cands[2][:N]


# speedup vs baseline: 1.0226x; 1.0199x over previous
import jax
import jax.numpy as jnp
from jax import lax
from jax.experimental import pallas as pl
from jax.experimental.pallas import tpu as pltpu
from jax.experimental.pallas import tpu_sc as plsc

N = 10000
E = 320000
NC = 2
NS = 16
NW = NC * NS
CHUNK = 128
NCHUNK = 80
P = CHUNK * NCHUNK
EP = P * NW
NP = 10240
ROWS_PER_TILE = NP // NS


_CS_COLS = 512
_CS_ROWS = EP // _CS_COLS
_CS_BLOCK_ROWS = 8
_CS_GRID = _CS_ROWS // _CS_BLOCK_ROWS


def _cos_sin_body(theta_ref, g_ref, cos_ref, sin_ref):
    i = pl.program_id(0)
    ang = theta_ref[...] + g_ref[...]
    ridx = lax.broadcasted_iota(jnp.int32, (_CS_BLOCK_ROWS, _CS_COLS), 0)
    cidx = lax.broadcasted_iota(jnp.int32, (_CS_BLOCK_ROWS, _CS_COLS), 1)
    flat = (i * _CS_BLOCK_ROWS + ridx) * _CS_COLS + cidx
    valid = flat < E
    cos_ref[...] = jnp.where(valid, jnp.cos(ang), 0.0)
    sin_ref[...] = jnp.where(valid, jnp.sin(ang), 0.0)


def _cos_sin(theta_p, g_p):
    spec = pl.BlockSpec((_CS_BLOCK_ROWS, _CS_COLS), lambda i: (i, 0))
    out = pl.pallas_call(
        _cos_sin_body,
        grid=(_CS_GRID,),
        in_specs=[spec, spec],
        out_specs=[spec, spec],
        out_shape=[
            jax.ShapeDtypeStruct((_CS_ROWS, _CS_COLS), jnp.float32),
            jax.ShapeDtypeStruct((_CS_ROWS, _CS_COLS), jnp.float32),
        ],
    )(theta_p.reshape(_CS_ROWS, _CS_COLS), g_p.reshape(_CS_ROWS, _CS_COLS))
    return out[0], out[1]


_MM_ROWS = 512
_MM_GRID = NP // _MM_ROWS


def _mm_body(h_ref, wself_ref, wcs_ref, self_ref, table_ref):
    h = h_ref[...]
    self_ref[...] = jnp.dot(h, wself_ref[...], preferred_element_type=jnp.float32)
    tab = jnp.dot(h, wcs_ref[...], preferred_element_type=jnp.float32)
    table_ref[...] = tab.astype(jnp.bfloat16)


def _row_spec(cols):
    return pl.BlockSpec((_MM_ROWS, cols), lambda i: (i, 0))


def _full_spec(r, c):
    return pl.BlockSpec((r, c), lambda i: (0, 0))


def _mm(h, w_self, w_cs):
    din = h.shape[1]
    dself = w_self.shape[1]
    dtab = w_cs.shape[1]
    return pl.pallas_call(
        _mm_body,
        grid=(_MM_GRID,),
        in_specs=[_row_spec(din), _full_spec(din, dself), _full_spec(din, dtab)],
        out_specs=[_row_spec(dself), _row_spec(dtab)],
        out_shape=[
            jax.ShapeDtypeStruct((NP, dself), jnp.float32),
            jax.ShapeDtypeStruct((NP, dtab), jnp.bfloat16),
        ],
    )(h, w_self, w_cs)


def _combine_body(selfp_ref, a00_ref, a01_ref, a10_ref, a11_ref, out_ref):
    lo = a00_ref[...] + a01_ref[...]
    hi = a10_ref[...] + a11_ref[...]
    agg = jnp.concatenate([lo, hi], axis=1)
    out_ref[...] = jnp.maximum(selfp_ref[...] + agg, 0.0)


def _combine(self_i, aggs):
    return pl.pallas_call(
        _combine_body,
        grid=(_MM_GRID,),
        in_specs=[_row_spec(128), _row_spec(64), _row_spec(64),
                  _row_spec(64), _row_spec(64)],
        out_specs=_row_spec(128),
        out_shape=jax.ShapeDtypeStruct((NP, 128), jnp.float32),
    )(self_i, aggs[0, 0], aggs[0, 1], aggs[1, 0], aggs[1, 1])


def _make_edge_kernel(dout):
    nred = dout // 16
    mesh = plsc.VectorSubcoreMesh(core_axis_name="c", subcore_axis_name="s")

    def body(table_hbm, src_hbm, dst_hbm, cos_hbm, sin_hbm, zeros_hbm, out_hbm,
             srcv, dstv, cosv, sinv, rows0, rows1, msg, agg, sem0, sem1):
        cid = lax.axis_index("c")
        sid = lax.axis_index("s")
        wid = sid * NC + cid

        pltpu.sync_copy(src_hbm.at[wid], srcv)
        pltpu.sync_copy(dst_hbm.at[wid], dstv)
        pltpu.sync_copy(cos_hbm.at[wid], cosv)
        pltpu.sync_copy(sin_hbm.at[wid], sinv)

        pltpu.sync_copy(zeros_hbm.at[pl.ds(sid * ROWS_PER_TILE, ROWS_PER_TILE)],
                        agg.at[pl.ds(sid * ROWS_PER_TILE, ROWS_PER_TILE)])
        plsc.subcore_barrier()

        himask = jnp.int32(-65536)

        def compute_scatter(j, rows):
            @plsc.parallel_loop(0, CHUNK // 16, unroll=2)
            def edge_body(q):
                cvec = cosv[j, pl.ds(q * 16, 16)]
                svec = sinv[j, pl.ds(q * 16, 16)]
                for i in range(16):
                    e = q * 16 + i
                    c = cvec[i]
                    s = svec[i]
                    for k in range(dout // 32):
                        wa = rows[e, pl.ds(16 * k, 16)]
                        wb = rows[e, pl.ds(dout // 2 + 16 * k, 16)]
                        a_lo = lax.bitcast_convert_type(wa << 16, jnp.float32)
                        a_hi = lax.bitcast_convert_type(wa & himask, jnp.float32)
                        b_lo = lax.bitcast_convert_type(wb << 16, jnp.float32)
                        b_hi = lax.bitcast_convert_type(wb & himask, jnp.float32)
                        msg[e, pl.ds(32 * k, 16)] = c * a_lo + s * b_lo
                        msg[e, pl.ds(32 * k + 16, 16)] = c * a_hi + s * b_hi
            pltpu.sync_copy(msg, agg.at[dstv.at[j]], add=True)

        pltpu.async_copy(table_hbm.at[srcv.at[0]], rows0, sem0)

        def chunk_pair(i, carry):
            j0 = 2 * i
            pltpu.async_copy(table_hbm.at[srcv.at[j0 + 1]], rows1, sem1)
            pltpu.make_async_copy(table_hbm.at[srcv.at[j0]], rows0, sem0).wait()
            compute_scatter(j0, rows0)

            @pl.when(j0 + 2 < NCHUNK)
            def _():
                pltpu.async_copy(table_hbm.at[srcv.at[j0 + 2]], rows0, sem0)

            pltpu.make_async_copy(
                table_hbm.at[srcv.at[j0 + 1]], rows1, sem1).wait()
            compute_scatter(j0 + 1, rows1)
            return carry

        lax.fori_loop(0, NCHUNK // 2, chunk_pair, 0)
        plsc.subcore_barrier()

        pltpu.sync_copy(agg.at[pl.ds(sid * ROWS_PER_TILE, ROWS_PER_TILE)],
                        out_hbm.at[cid, pl.ds(sid * ROWS_PER_TILE, ROWS_PER_TILE)])

    return pl.kernel(
        body,
        out_type=jax.ShapeDtypeStruct((NC, NP, dout), jnp.float32),
        mesh=mesh,
        scratch_types=[
            pltpu.VMEM((NCHUNK, CHUNK), jnp.int32),
            pltpu.VMEM((NCHUNK, CHUNK), jnp.int32),
            pltpu.VMEM((NCHUNK, CHUNK), jnp.float32),
            pltpu.VMEM((NCHUNK, CHUNK), jnp.float32),
            pltpu.VMEM((CHUNK, dout), jnp.int32),
            pltpu.VMEM((CHUNK, dout), jnp.int32),
            pltpu.VMEM((CHUNK, dout), jnp.float32),
            pltpu.VMEM_SHARED((NP, dout), jnp.float32),
            pltpu.SemaphoreType.DMA,
            pltpu.SemaphoreType.DMA,
        ],
        compiler_params=pltpu.CompilerParams(use_tc_tiling_on_sc=False),
    )


def _pad_rows(w, rows):
    return jnp.concatenate([w, jnp.zeros((rows - w.shape[0], w.shape[1]),
                                         jnp.float32)], axis=0)


def kernel(pos, x, edge_index, theta, g,
           W0_self, W0_c, W0_s, W1_self, W1_c, W1_s, W2_self, W2_c, W2_s):
    h0 = jnp.concatenate([pos, x], axis=1)
    h0 = jnp.concatenate([h0, jnp.zeros((NP - N, 128), jnp.float32)], axis=0)
    src = edge_index[0]
    dst = edge_index[1]

    pad = EP - E
    padi = jnp.zeros((pad,), jnp.int32)
    padf = jnp.zeros((pad,), jnp.float32)
    src_p = jnp.concatenate([src, padi]).reshape(NW, NCHUNK, CHUNK)
    dst_p = jnp.concatenate([dst, padi]).reshape(NW, NCHUNK, CHUNK)
    theta_p = jnp.concatenate([theta, padf])
    g_p = jnp.concatenate([g, padf])

    cos_f, sin_f = _cos_sin(theta_p, g_p)
    cos_p = cos_f.reshape(NW, NCHUNK, CHUNK)
    sin_p = sin_f.reshape(NW, NCHUNK, CHUNK)

    zeros64 = jnp.zeros((NP, 64), jnp.float32)
    edge64 = _make_edge_kernel(64)

    z32 = jnp.zeros((96, 32), jnp.float32)

    perm = []
    for k in (0, 1):
        for i in range(16):
            perm.extend([32 * k + i, 32 * k + 16 + i])
    perm = jnp.array(perm, jnp.int32)

    def cs_pair(wc, ws, lo, hi, width):
        zc = jnp.zeros((wc.shape[0], 64 - width), jnp.float32)
        yc = jnp.concatenate([wc[:, lo:hi], zc], axis=1)[:, perm]
        ys = jnp.concatenate([ws[:, lo:hi], zc], axis=1)[:, perm]
        return jnp.concatenate([yc, ys], axis=1)

    wself = jnp.stack([
        jnp.concatenate([W0_self, jnp.zeros((128, 32), jnp.float32)], axis=1),
        _pad_rows(jnp.concatenate([W1_self, z32], axis=1), 128),
        _pad_rows(W2_self, 128),
    ])
    wcs = jnp.stack([
        jnp.concatenate([cs_pair(W0_c, W0_s, 0, 64, 64),
                         cs_pair(W0_c, W0_s, 64, 96, 32)], axis=1),
        _pad_rows(jnp.concatenate([cs_pair(W1_c, W1_s, 0, 64, 64),
                                   cs_pair(W1_c, W1_s, 64, 96, 32)], axis=1),
                  128),
        _pad_rows(jnp.concatenate([cs_pair(W2_c, W2_s, 0, 64, 64),
                                   cs_pair(W2_c, W2_s, 64, 128, 64)], axis=1),
                  128),
    ])

    def pass_body(carry, table_p):
        agg = edge64(table_p, src_p, dst_p, cos_p, sin_p, zeros64)
        return carry, agg

    def layer_body(carry, ws):
        h, k = carry
        w_self, w_cs = ws
        self_i, tabflat = _mm(h, w_self, w_cs)
        tab32 = lax.bitcast_convert_type(
            tabflat.reshape(NP, 128, 2), jnp.int32)
        tables = jnp.stack([tab32[:, :64], tab32[:, 64:]])
        _, aggs = lax.scan(pass_body, 0, tables)
        cand = _combine(self_i, aggs)
        h_next = jnp.where(k < 2, cand, h)
        return (h_next, k + 1), cand

    (_, _), cands = lax.scan(layer_body, (h0, jnp.int32(0)), (wself, wcs))
    return cands[2][:N]


# speedup vs baseline: 1.1050x; 1.0806x over previous
import jax
import jax.numpy as jnp
from jax import lax
from jax.experimental import pallas as pl
from jax.experimental.pallas import tpu as pltpu
from jax.experimental.pallas import tpu_sc as plsc

N = 10000
E = 320000
NC = 2
NS = 16
NW = NC * NS
CHUNK = 128
NCH0 = 110
NCH1 = 50
NCHSUM = NCH0 + NCH1
MAXCH = max(NCH0, NCH1)
EP = NS * NCHSUM * CHUNK
NP = 10240
ROWS_PER_TILE = NP // NS


_CS_COLS = 512
_CS_ROWS = EP // _CS_COLS
_CS_BLOCK_ROWS = 8
_CS_GRID = _CS_ROWS // _CS_BLOCK_ROWS


def _cos_sin_body(theta_ref, g_ref, cos_ref, sin_ref):
    i = pl.program_id(0)
    ang = theta_ref[...] + g_ref[...]
    ridx = lax.broadcasted_iota(jnp.int32, (_CS_BLOCK_ROWS, _CS_COLS), 0)
    cidx = lax.broadcasted_iota(jnp.int32, (_CS_BLOCK_ROWS, _CS_COLS), 1)
    flat = (i * _CS_BLOCK_ROWS + ridx) * _CS_COLS + cidx
    valid = flat < E
    cos_ref[...] = jnp.where(valid, jnp.cos(ang), 0.0)
    sin_ref[...] = jnp.where(valid, jnp.sin(ang), 0.0)


def _cos_sin(theta_p, g_p):
    spec = pl.BlockSpec((_CS_BLOCK_ROWS, _CS_COLS), lambda i: (i, 0))
    out = pl.pallas_call(
        _cos_sin_body,
        grid=(_CS_GRID,),
        in_specs=[spec, spec],
        out_specs=[spec, spec],
        out_shape=[
            jax.ShapeDtypeStruct((_CS_ROWS, _CS_COLS), jnp.float32),
            jax.ShapeDtypeStruct((_CS_ROWS, _CS_COLS), jnp.float32),
        ],
    )(theta_p.reshape(_CS_ROWS, _CS_COLS), g_p.reshape(_CS_ROWS, _CS_COLS))
    return out[0], out[1]


_MM_ROWS = 512
_MM_GRID = NP // _MM_ROWS


def _mm_body(h_ref, wself_ref, wcs_ref, self_ref, table_ref):
    h = h_ref[...]
    self_ref[...] = jnp.dot(h, wself_ref[...], preferred_element_type=jnp.float32)
    tab = jnp.dot(h, wcs_ref[...], preferred_element_type=jnp.float32)
    table_ref[...] = tab.astype(jnp.bfloat16)


def _row_spec(cols):
    return pl.BlockSpec((_MM_ROWS, cols), lambda i: (i, 0))


def _full_spec(r, c):
    return pl.BlockSpec((r, c), lambda i: (0, 0))


def _mm(h, w_self, w_cs):
    din = h.shape[1]
    dself = w_self.shape[1]
    dtab = w_cs.shape[1]
    return pl.pallas_call(
        _mm_body,
        grid=(_MM_GRID,),
        in_specs=[_row_spec(din), _full_spec(din, dself), _full_spec(din, dtab)],
        out_specs=[_row_spec(dself), _row_spec(dtab)],
        out_shape=[
            jax.ShapeDtypeStruct((NP, dself), jnp.float32),
            jax.ShapeDtypeStruct((NP, dtab), jnp.bfloat16),
        ],
    )(h, w_self, w_cs)


def _combine_body(selfp_ref, a00_ref, a01_ref, a10_ref, a11_ref, out_ref):
    lo = a00_ref[...] + a01_ref[...]
    hi = a10_ref[...] + a11_ref[...]
    agg = jnp.concatenate([lo, hi], axis=1)
    out_ref[...] = jnp.maximum(selfp_ref[...] + agg, 0.0)


def _combine(self_i, aggs):
    return pl.pallas_call(
        _combine_body,
        grid=(_MM_GRID,),
        in_specs=[_row_spec(128), _row_spec(64), _row_spec(64),
                  _row_spec(64), _row_spec(64)],
        out_specs=_row_spec(128),
        out_shape=jax.ShapeDtypeStruct((NP, 128), jnp.float32),
    )(self_i, aggs[0, 0], aggs[0, 1], aggs[1, 0], aggs[1, 1])


def _make_edge_kernel(dout):
    nred = dout // 16
    mesh = plsc.VectorSubcoreMesh(core_axis_name="c", subcore_axis_name="s")

    def body(table_hbm, src_hbm, dst_hbm, cos_hbm, sin_hbm, zeros_hbm, out_hbm,
             srcv, dstv, cosv, sinv, rows0, rows1, msg, agg, sem0, sem1):
        cid = lax.axis_index("c")
        sid = lax.axis_index("s")
        wid = sid * NC + cid
        nch = jnp.where(cid == 0, NCH0, NCH1)

        pltpu.sync_copy(src_hbm.at[wid], srcv)
        pltpu.sync_copy(dst_hbm.at[wid], dstv)
        pltpu.sync_copy(cos_hbm.at[wid], cosv)
        pltpu.sync_copy(sin_hbm.at[wid], sinv)

        pltpu.sync_copy(zeros_hbm.at[pl.ds(sid * ROWS_PER_TILE, ROWS_PER_TILE)],
                        agg.at[pl.ds(sid * ROWS_PER_TILE, ROWS_PER_TILE)])
        plsc.subcore_barrier()

        himask = jnp.int32(-65536)

        def compute_scatter(j, rows):
            @plsc.parallel_loop(0, CHUNK // 16, unroll=2)
            def edge_body(q):
                cvec = cosv[j, pl.ds(q * 16, 16)]
                svec = sinv[j, pl.ds(q * 16, 16)]
                for i in range(16):
                    e = q * 16 + i
                    c = cvec[i]
                    s = svec[i]
                    for k in range(dout // 32):
                        wa = rows[e, pl.ds(16 * k, 16)]
                        wb = rows[e, pl.ds(dout // 2 + 16 * k, 16)]
                        a_lo = lax.bitcast_convert_type(wa << 16, jnp.float32)
                        a_hi = lax.bitcast_convert_type(wa & himask, jnp.float32)
                        b_lo = lax.bitcast_convert_type(wb << 16, jnp.float32)
                        b_hi = lax.bitcast_convert_type(wb & himask, jnp.float32)
                        msg[e, pl.ds(32 * k, 16)] = c * a_lo + s * b_lo
                        msg[e, pl.ds(32 * k + 16, 16)] = c * a_hi + s * b_hi
            pltpu.sync_copy(msg, agg.at[dstv.at[j]], add=True)

        pltpu.async_copy(table_hbm.at[srcv.at[0]], rows0, sem0)

        def chunk_pair(i, carry):
            j0 = 2 * i
            pltpu.async_copy(table_hbm.at[srcv.at[j0 + 1]], rows1, sem1)
            pltpu.make_async_copy(table_hbm.at[srcv.at[j0]], rows0, sem0).wait()
            compute_scatter(j0, rows0)

            @pl.when(j0 + 2 < nch)
            def _():
                pltpu.async_copy(table_hbm.at[srcv.at[j0 + 2]], rows0, sem0)

            pltpu.make_async_copy(
                table_hbm.at[srcv.at[j0 + 1]], rows1, sem1).wait()
            compute_scatter(j0 + 1, rows1)
            return carry

        lax.fori_loop(0, nch // 2, chunk_pair, 0)
        plsc.subcore_barrier()

        pltpu.sync_copy(agg.at[pl.ds(sid * ROWS_PER_TILE, ROWS_PER_TILE)],
                        out_hbm.at[cid, pl.ds(sid * ROWS_PER_TILE, ROWS_PER_TILE)])

    return pl.kernel(
        body,
        out_type=jax.ShapeDtypeStruct((NC, NP, dout), jnp.float32),
        mesh=mesh,
        scratch_types=[
            pltpu.VMEM((MAXCH, CHUNK), jnp.int32),
            pltpu.VMEM((MAXCH, CHUNK), jnp.int32),
            pltpu.VMEM((MAXCH, CHUNK), jnp.float32),
            pltpu.VMEM((MAXCH, CHUNK), jnp.float32),
            pltpu.VMEM((CHUNK, dout), jnp.int32),
            pltpu.VMEM((CHUNK, dout), jnp.int32),
            pltpu.VMEM((CHUNK, dout), jnp.float32),
            pltpu.VMEM_SHARED((NP, dout), jnp.float32),
            pltpu.SemaphoreType.DMA,
            pltpu.SemaphoreType.DMA,
        ],
        compiler_params=pltpu.CompilerParams(use_tc_tiling_on_sc=False),
    )


def _pad_rows(w, rows):
    return jnp.concatenate([w, jnp.zeros((rows - w.shape[0], w.shape[1]),
                                         jnp.float32)], axis=0)


def kernel(pos, x, edge_index, theta, g,
           W0_self, W0_c, W0_s, W1_self, W1_c, W1_s, W2_self, W2_c, W2_s):
    h0 = jnp.concatenate([pos, x], axis=1)
    h0 = jnp.concatenate([h0, jnp.zeros((NP - N, 128), jnp.float32)], axis=0)
    src = edge_index[0]
    dst = edge_index[1]

    pad = EP - E
    padi = jnp.zeros((pad,), jnp.int32)
    padf = jnp.zeros((pad,), jnp.float32)
    theta_p = jnp.concatenate([theta, padf])
    g_p = jnp.concatenate([g, padf])
    cos_f, sin_f = _cos_sin(theta_p, g_p)

    def split_uneven(a):
        a = a.reshape(NS, NCHSUM * CHUNK)
        a0 = a[:, :NCH0 * CHUNK].reshape(NS, NCH0, CHUNK)
        a1 = a[:, NCH0 * CHUNK:].reshape(NS, NCH1, CHUNK)
        z = jnp.zeros((NS, MAXCH, CHUNK), a.dtype)
        a0 = jnp.concatenate([a0, z[:, :MAXCH - NCH0]], axis=1)
        a1 = jnp.concatenate([a1, z[:, :MAXCH - NCH1]], axis=1)
        return jnp.stack([a0, a1], axis=1).reshape(NW, MAXCH, CHUNK)

    src_p = split_uneven(jnp.concatenate([src, padi]))
    dst_p = split_uneven(jnp.concatenate([dst, padi]))
    cos_p = split_uneven(cos_f.reshape(EP))
    sin_p = split_uneven(sin_f.reshape(EP))

    zeros64 = jnp.zeros((NP, 64), jnp.float32)
    edge64 = _make_edge_kernel(64)

    z32 = jnp.zeros((96, 32), jnp.float32)

    perm = []
    for k in (0, 1):
        for i in range(16):
            perm.extend([32 * k + i, 32 * k + 16 + i])
    perm = jnp.array(perm, jnp.int32)

    def cs_pair(wc, ws, lo, hi, width):
        zc = jnp.zeros((wc.shape[0], 64 - width), jnp.float32)
        yc = jnp.concatenate([wc[:, lo:hi], zc], axis=1)[:, perm]
        ys = jnp.concatenate([ws[:, lo:hi], zc], axis=1)[:, perm]
        return jnp.concatenate([yc, ys], axis=1)

    wself = jnp.stack([
        jnp.concatenate([W0_self, jnp.zeros((128, 32), jnp.float32)], axis=1),
        _pad_rows(jnp.concatenate([W1_self, z32], axis=1), 128),
        _pad_rows(W2_self, 128),
    ])
    wcs = jnp.stack([
        jnp.concatenate([cs_pair(W0_c, W0_s, 0, 64, 64),
                         cs_pair(W0_c, W0_s, 64, 96, 32)], axis=1),
        _pad_rows(jnp.concatenate([cs_pair(W1_c, W1_s, 0, 64, 64),
                                   cs_pair(W1_c, W1_s, 64, 96, 32)], axis=1),
                  128),
        _pad_rows(jnp.concatenate([cs_pair(W2_c, W2_s, 0, 64, 64),
                                   cs_pair(W2_c, W2_s, 64, 128, 64)], axis=1),
                  128),
    ])

    def pass_body(carry, table_p):
        agg = edge64(table_p, src_p, dst_p, cos_p, sin_p, zeros64)
        return carry, agg

    def layer_body(carry, ws):
        h, k = carry
        w_self, w_cs = ws
        self_i, tabflat = _mm(h, w_self, w_cs)
        tab32 = lax.bitcast_convert_type(
            tabflat.reshape(NP, 128, 2), jnp.int32)
        tables = jnp.stack([tab32[:, :64], tab32[:, 64:]])
        _, aggs = lax.scan(pass_body, 0, tables)
        cand = _combine(self_i, aggs)
        h_next = jnp.where(k < 2, cand, h)
        return (h_next, k + 1), cand

    (_, _), cands = lax.scan(layer_body, (h0, jnp.int32(0)), (wself, wcs))
    return cands[2][:N]
